# baseline (speedup 1.0000x reference)
import functools

import jax
import jax.numpy as jnp
from jax import lax
from jax.experimental import pallas as pl
from jax.experimental.pallas import tpu as pltpu
from jax.experimental.pallas import tpu_sc as plsc

B = 4096
L = 200
D = 32
V = 1000000
NW = 32
CHUNK = 512
NQ = B // CHUNK
ITEMS = L * NQ
PER_W = ITEMS // NW
NG = CHUNK // 128
TBLK = 512
VFULL = V // TBLK
VMAIN = (VFULL // NW) * NW
KMAIN = VMAIN // NW
VTAIL = V - VFULL * TBLK

_mesh = plsc.VectorSubcoreMesh(core_axis_name="c", subcore_axis_name="s")


@functools.partial(
    pl.kernel,
    out_type=(
        jax.ShapeDtypeStruct((V // 4, 128), jnp.float32),
        jax.ShapeDtypeStruct((L, B // 128, 128), jnp.int32),
    ),
    mesh=_mesh,
    scratch_types=[
        pltpu.VMEM((2, 32, TBLK + 1), jnp.float32),
        pltpu.VMEM((2, TBLK // 4, 128), jnp.float32),
        pltpu.VMEM((PER_W // 2, 8, 128), jnp.int32),
        pltpu.SemaphoreType.DMA,
        pltpu.SemaphoreType.DMA,
        pltpu.SemaphoreType.DMA,
        pltpu.SemaphoreType.DMA,
        pltpu.SemaphoreType.DMA,
        pltpu.SemaphoreType.DMA,
    ],
    compiler_params=pltpu.CompilerParams(use_tc_tiling_on_sc=True,
                                         needs_layout_passes=False),
)
def _prep(xt_hbm, tabt_hbm, tail_hbm, trm_hbm, xi_hbm,
          tin_v, tout_v, xb_v,
          isem0, isem1, osem0, osem1, xisem, xosem):
    wid = lax.axis_index("s") * 2 + lax.axis_index("c")
    iota = lax.iota(jnp.int32, 16)
    isem = (isem0, isem1)
    osem = (osem0, osem1)

    def transpose_block(src, dst):
        def body(t, _):
            col = jnp.full((16,), t, jnp.int32)
            lo = plsc.load_gather(src, [iota, col])
            hi = plsc.load_gather(src, [iota + 16, col])
            r = t // 4
            c0 = (t % 4) * 32
            dst[r, pl.ds(c0, 16)] = lo
            dst[r, pl.ds(c0 + 16, 16)] = hi
            return 0
        lax.fori_loop(0, TBLK, body, 0, unroll=16)

    def start_in(k, buf):
        b = wid + NW * k
        pltpu.async_copy(tabt_hbm.at[:, pl.ds(b * TBLK, TBLK)],
                         tin_v.at[buf, :, pl.ds(0, TBLK)], isem[buf])

    def wait_in(buf):
        pltpu.make_async_copy(tabt_hbm.at[:, pl.ds(0, TBLK)],
                              tin_v.at[buf, :, pl.ds(0, TBLK)],
                              isem[buf]).wait()

    def start_out(k, buf):
        b = wid + NW * k
        pltpu.async_copy(tout_v.at[buf],
                         trm_hbm.at[pl.ds(b * (TBLK // 4), TBLK // 4)],
                         osem[buf])

    def wait_out(k, buf):
        b = wid + NW * k
        pltpu.make_async_copy(tout_v.at[buf],
                              trm_hbm.at[pl.ds(b * (TBLK // 4), TBLK // 4)],
                              osem[buf]).wait()

    start_in(0, 0)
    start_in(1, 1)

    def pair_body(kk, carry):
        for par in (0, 1):
            k = kk * 2 + par
            wait_in(par)

            @pl.when(k >= 2)
            def _():
                wait_out(k - 2, par)

            transpose_block(tin_v.at[par], tout_v.at[par])

            @pl.when(k + 2 < KMAIN)
            def _():
                start_in(k + 2, par)

            start_out(k, par)
        return carry

    lax.fori_loop(0, KMAIN // 2, pair_body, 0)
    wait_in(0)
    wait_out(KMAIN - 3, 0)
    transpose_block(tin_v.at[0], tout_v.at[0])
    start_out(KMAIN - 1, 0)
    wait_out(KMAIN - 2, 1)
    wait_out(KMAIN - 1, 0)

    @pl.when(wid < VFULL - VMAIN)
    def _ragged():
        b = VMAIN + wid
        pltpu.sync_copy(tabt_hbm.at[:, pl.ds(b * TBLK, TBLK)],
                        tin_v.at[0, :, pl.ds(0, TBLK)])
        transpose_block(tin_v.at[0], tout_v.at[0])
        pltpu.sync_copy(tout_v.at[0],
                        trm_hbm.at[pl.ds(b * (TBLK // 4), TBLK // 4)])

    @pl.when(wid == VFULL - VMAIN)
    def _tail():
        pltpu.sync_copy(tail_hbm, tout_v.at[0, pl.ds(0, VTAIL * D // 128)])
        pltpu.sync_copy(tout_v.at[0, pl.ds(0, VTAIL * D // 128)],
                        trm_hbm.at[pl.ds(VFULL * (TBLK // 4),
                                         VTAIL * D // 128)])

    NB = PER_W // 2
    for i in range(NB):
        m = wid * NB + i
        lt = m // (B // 128)
        c = m % (B // 128)
        pltpu.async_copy(xt_hbm.at[pl.ds(lt * 8, 8), pl.ds(c * 128, 128)],
                         xb_v.at[i], xisem)
    for i in range(NB):
        pltpu.make_async_copy(xt_hbm.at[pl.ds(0, 8), pl.ds(0, 128)],
                              xb_v.at[i], xisem).wait()
    for i in range(NB):
        m = wid * NB + i
        lt = m // (B // 128)
        c = m % (B // 128)
        pltpu.async_copy(xb_v.at[i], xi_hbm.at[pl.ds(lt * 8, 8), c], xosem)
    for i in range(NB):
        m = wid * NB + i
        lt = m // (B // 128)
        c = m % (B // 128)
        pltpu.make_async_copy(xb_v.at[i], xi_hbm.at[pl.ds(lt * 8, 8), c],
                              xosem).wait()


@functools.partial(
    pl.kernel,
    out_type=jax.ShapeDtypeStruct((L, D // 8, B // 128, 8, 128), jnp.float32),
    mesh=_mesh,
    scratch_types=[
        pltpu.VMEM((4, NG, 128), jnp.int32),
        pltpu.VMEM((4, CHUNK, D), jnp.float32),
        pltpu.VMEM((4, 4, 128), jnp.float32),
        pltpu.VMEM((2, D // 8, NG, 8, 128), jnp.float32),
        pltpu.SemaphoreType.DMA,
        pltpu.SemaphoreType.DMA,
        pltpu.SemaphoreType.DMA,
        pltpu.SemaphoreType.DMA,
        pltpu.SemaphoreType.DMA,
        pltpu.SemaphoreType.DMA,
        pltpu.SemaphoreType.DMA,
        pltpu.SemaphoreType.DMA,
        pltpu.SemaphoreType.DMA,
        pltpu.SemaphoreType.DMA,
    ],
    compiler_params=pltpu.CompilerParams(use_tc_tiling_on_sc=False,
                                         needs_layout_passes=False),
)
def _lookup(xi_hbm, trm_hbm, posb_hbm, out_hbm,
            idx_v, rows_v, pos_v, ob_v,
            isem0, isem1, isem2, isem3,
            gsem0, gsem1, gsem2, gsem3, osemA, osemB):
    wid = lax.axis_index("s") * 2 + lax.axis_index("c")
    iota = lax.iota(jnp.int32, 16)
    isem = (isem0, isem1, isem2, isem3)
    gsem = (gsem0, gsem1, gsem2, gsem3)
    osem = (osemA, osemB)

    def lq(j):
        m = wid * PER_W + j
        return m // NQ, m % NQ

    def start_inputs(j, par):
        l, q = lq(j)
        pltpu.async_copy(xi_hbm.at[l, pl.ds(q * NG, NG)], idx_v.at[par],
                         isem[par])
        pltpu.async_copy(posb_hbm.at[pl.ds(l * 4, 4)], pos_v.at[par],
                         isem[par])

    def wait_inputs(par):
        pltpu.make_async_copy(xi_hbm.at[0, pl.ds(0, NG)], idx_v.at[par],
                              isem[par]).wait()
        pltpu.make_async_copy(posb_hbm.at[pl.ds(0, 4)], pos_v.at[par],
                              isem[par]).wait()

    def start_gathers(par):
        for k in range(NG):
            pltpu.async_copy(trm_hbm.at[idx_v.at[par, k]],
                             rows_v.at[par, pl.ds(k * 128, 128)], gsem[par])

    def wait_gathers(par):
        pltpu.make_async_copy(trm_hbm.at[pl.ds(0, CHUNK)], rows_v.at[par],
                              gsem[par]).wait()

    def start_write(j, par):
        l, q = lq(j)
        pltpu.async_copy(ob_v.at[par],
                         out_hbm.at[l, :, pl.ds(q * NG, NG)], osem[par])

    def wait_write(j, par):
        l, q = lq(j)
        pltpu.make_async_copy(ob_v.at[par],
                              out_hbm.at[l, :, pl.ds(q * NG, NG)],
                              osem[par]).wait()

    perm_lo = [(iota - (1 << k)) & 15 for k in range(4)]
    perm_hi = [(iota + (1 << k)) & 15 for k in range(4)]
    masks = [(iota & (1 << k)) == 0 for k in range(4)]
    _dnums = lax.GatherDimensionNumbers(
        offset_dims=(), collapsed_slice_dims=(0,), start_index_map=(0,))

    def _shift(v, perm):
        return lax.gather(v, perm[:, None], _dnums, (1,),
                          mode=lax.GatherScatterMode.PROMISE_IN_BOUNDS)

    def compute(par, p2):
        for dh in range(2):
            def g_body(g, _, dh=dh):
                t0 = g * 16
                cur = [rows_v[par, t0 + i, pl.ds(dh * 16, 16)]
                       for i in range(16)]
                for k in range(4):
                    m = 1 << k
                    nxt = [None] * 16
                    for i in range(16):
                        if i & m == 0:
                            sh = _shift(cur[i + m], perm_lo[k])
                            nxt[i] = jnp.where(masks[k], cur[i], sh)
                        else:
                            sh = _shift(cur[i - m], perm_hi[k])
                            nxt[i] = jnp.where(masks[k], sh, cur[i])
                    cur = nxt
                cp = g // 8
                mm = g % 8
                for j in range(16):
                    d = dh * 16 + j
                    splat = pos_v[par, d // 8, pl.ds((d % 8) * 16, 16)]
                    ob_v[p2, d // 8, cp, d % 8, pl.ds(mm * 16, 16)] = (
                        cur[j] + splat)
                return 0

            lax.fori_loop(0, CHUNK // 16, g_body, 0)

    start_inputs(0, 0)
    start_inputs(1, 1)
    start_inputs(2, 2)
    wait_inputs(0)
    start_gathers(0)
    wait_inputs(1)
    start_gathers(1)

    def quad_body(kk, carry):
        for par in range(4):
            j = kk * 4 + par
            p2 = par % 2
            wait_inputs((par + 2) % 4)
            start_gathers((par + 2) % 4)
            wait_gathers(par)

            @pl.when(j >= 2)
            def _():
                wait_write(j - 2, p2)

            compute(par, p2)
            start_write(j, p2)

            @pl.when(j + 3 < PER_W)
            def _():
                start_inputs(j + 3, (par + 3) % 4)
        return carry

    lax.fori_loop(0, (PER_W - 2) // 4, quad_body, 0)
    for j in (PER_W - 2, PER_W - 1):
        par = j % 4
        p2 = j % 2
        wait_gathers(par)
        wait_write(j - 2, p2)
        compute(par, p2)
        start_write(j, p2)
    wait_write(PER_W - 2, 0)
    wait_write(PER_W - 1, 1)


def kernel(x, embedding_table, possitional_emb):
    xt = x.T.astype(jnp.int32)
    tabt = embedding_table.T
    tail = embedding_table[VFULL * TBLK:].reshape(VTAIL * D // 128, 128)
    posb = (jnp.broadcast_to(possitional_emb[:, :, None], (L, D, 16))
            .reshape(L * 4, 128))
    trm, xi = _prep(xt, tabt, tail)
    out5 = _lookup(xi, trm.reshape(V, D), posb)
    return out5.transpose(2, 4, 0, 1, 3).reshape(B, L, D)


# speedup vs baseline: 1.2393x; 1.2393x over previous
import functools

import jax
import jax.numpy as jnp
from jax import lax
from jax.experimental import pallas as pl
from jax.experimental.pallas import tpu as pltpu
from jax.experimental.pallas import tpu_sc as plsc

B = 4096
L = 200
D = 32
V = 1000000
NW = 32
CHUNK = 512
NQ = B // CHUNK
ITEMS = L * NQ
PER_W = ITEMS // NW
NG = CHUNK // 128
TBLK = 512
VFULL = V // TBLK
VMAIN = (VFULL // NW) * NW
KMAIN = VMAIN // NW
VTAIL = V - VFULL * TBLK

_mesh = plsc.VectorSubcoreMesh(core_axis_name="c", subcore_axis_name="s")


@functools.partial(
    pl.kernel,
    out_type=jax.ShapeDtypeStruct((L, B // 128, 128), jnp.int32),
    mesh=_mesh,
    scratch_types=[
        pltpu.VMEM((PER_W // 2, 8, 128), jnp.int32),
        pltpu.SemaphoreType.DMA,
        pltpu.SemaphoreType.DMA,
    ],
    compiler_params=pltpu.CompilerParams(use_tc_tiling_on_sc=True,
                                         needs_layout_passes=False),
)
def _prep(xt_hbm, xi_hbm, xb_v, xisem, xosem):
    wid = lax.axis_index("s") * 2 + lax.axis_index("c")

    NB = PER_W // 2
    for i in range(NB):
        m = wid * NB + i
        lt = m // (B // 128)
        c = m % (B // 128)
        pltpu.async_copy(xt_hbm.at[pl.ds(lt * 8, 8), pl.ds(c * 128, 128)],
                         xb_v.at[i], xisem)
    for i in range(NB):
        pltpu.make_async_copy(xt_hbm.at[pl.ds(0, 8), pl.ds(0, 128)],
                              xb_v.at[i], xisem).wait()
    for i in range(NB):
        m = wid * NB + i
        lt = m // (B // 128)
        c = m % (B // 128)
        pltpu.async_copy(xb_v.at[i], xi_hbm.at[pl.ds(lt * 8, 8), c], xosem)
    for i in range(NB):
        m = wid * NB + i
        lt = m // (B // 128)
        c = m % (B // 128)
        pltpu.make_async_copy(xb_v.at[i], xi_hbm.at[pl.ds(lt * 8, 8), c],
                              xosem).wait()


@functools.partial(
    pl.kernel,
    out_type=jax.ShapeDtypeStruct((L, D // 8, B // 128, 8, 128), jnp.float32),
    mesh=_mesh,
    scratch_types=[
        pltpu.VMEM((4, NG, 128), jnp.int32),
        pltpu.VMEM((4, CHUNK, D), jnp.float32),
        pltpu.VMEM((4, 4, 128), jnp.float32),
        pltpu.VMEM((2, D // 8, NG, 8, 128), jnp.float32),
        pltpu.SemaphoreType.DMA,
        pltpu.SemaphoreType.DMA,
        pltpu.SemaphoreType.DMA,
        pltpu.SemaphoreType.DMA,
        pltpu.SemaphoreType.DMA,
        pltpu.SemaphoreType.DMA,
        pltpu.SemaphoreType.DMA,
        pltpu.SemaphoreType.DMA,
        pltpu.SemaphoreType.DMA,
        pltpu.SemaphoreType.DMA,
    ],
    compiler_params=pltpu.CompilerParams(use_tc_tiling_on_sc=False,
                                         needs_layout_passes=False),
)
def _lookup(xi_hbm, trm_hbm, posb_hbm, out_hbm,
            idx_v, rows_v, pos_v, ob_v,
            isem0, isem1, isem2, isem3,
            gsem0, gsem1, gsem2, gsem3, osemA, osemB):
    wid = lax.axis_index("s") * 2 + lax.axis_index("c")
    iota = lax.iota(jnp.int32, 16)
    isem = (isem0, isem1, isem2, isem3)
    gsem = (gsem0, gsem1, gsem2, gsem3)
    osem = (osemA, osemB)

    def lq(j):
        m = wid * PER_W + j
        return m // NQ, m % NQ

    def start_inputs(j, par):
        l, q = lq(j)
        pltpu.async_copy(xi_hbm.at[l, pl.ds(q * NG, NG)], idx_v.at[par],
                         isem[par])
        pltpu.async_copy(posb_hbm.at[pl.ds(l * 4, 4)], pos_v.at[par],
                         isem[par])

    def wait_inputs(par):
        pltpu.make_async_copy(xi_hbm.at[0, pl.ds(0, NG)], idx_v.at[par],
                              isem[par]).wait()
        pltpu.make_async_copy(posb_hbm.at[pl.ds(0, 4)], pos_v.at[par],
                              isem[par]).wait()

    def start_gathers(par):
        for k in range(NG):
            pltpu.async_copy(trm_hbm.at[idx_v.at[par, k]],
                             rows_v.at[par, pl.ds(k * 128, 128)], gsem[par])

    def wait_gathers(par):
        pltpu.make_async_copy(trm_hbm.at[pl.ds(0, CHUNK)], rows_v.at[par],
                              gsem[par]).wait()

    def start_write(j, par):
        l, q = lq(j)
        pltpu.async_copy(ob_v.at[par],
                         out_hbm.at[l, :, pl.ds(q * NG, NG)], osem[par])

    def wait_write(j, par):
        l, q = lq(j)
        pltpu.make_async_copy(ob_v.at[par],
                              out_hbm.at[l, :, pl.ds(q * NG, NG)],
                              osem[par]).wait()

    perm_lo = [(iota - (1 << k)) & 15 for k in range(4)]
    perm_hi = [(iota + (1 << k)) & 15 for k in range(4)]
    masks = [(iota & (1 << k)) == 0 for k in range(4)]
    _dnums = lax.GatherDimensionNumbers(
        offset_dims=(), collapsed_slice_dims=(0,), start_index_map=(0,))

    def _shift(v, perm):
        return lax.gather(v, perm[:, None], _dnums, (1,),
                          mode=lax.GatherScatterMode.PROMISE_IN_BOUNDS)

    def compute(par, p2):
        for dh in range(2):
            def g_body(g, _, dh=dh):
                t0 = g * 16
                cur = [rows_v[par, t0 + i, pl.ds(dh * 16, 16)]
                       for i in range(16)]
                for k in range(4):
                    m = 1 << k
                    nxt = [None] * 16
                    for i in range(16):
                        if i & m == 0:
                            sh = _shift(cur[i + m], perm_lo[k])
                            nxt[i] = jnp.where(masks[k], cur[i], sh)
                        else:
                            sh = _shift(cur[i - m], perm_hi[k])
                            nxt[i] = jnp.where(masks[k], sh, cur[i])
                    cur = nxt
                cp = g // 8
                mm = g % 8
                for j in range(16):
                    d = dh * 16 + j
                    splat = pos_v[par, d // 8, pl.ds((d % 8) * 16, 16)]
                    ob_v[p2, d // 8, cp, d % 8, pl.ds(mm * 16, 16)] = (
                        cur[j] + splat)
                return 0

            lax.fori_loop(0, CHUNK // 16, g_body, 0)

    start_inputs(0, 0)
    start_inputs(1, 1)
    start_inputs(2, 2)
    wait_inputs(0)
    start_gathers(0)
    wait_inputs(1)
    start_gathers(1)

    def quad_body(kk, carry):
        for par in range(4):
            j = kk * 4 + par
            p2 = par % 2
            wait_inputs((par + 2) % 4)
            start_gathers((par + 2) % 4)
            wait_gathers(par)

            @pl.when(j >= 2)
            def _():
                wait_write(j - 2, p2)

            compute(par, p2)
            start_write(j, p2)

            @pl.when(j + 3 < PER_W)
            def _():
                start_inputs(j + 3, (par + 3) % 4)
        return carry

    lax.fori_loop(0, (PER_W - 2) // 4, quad_body, 0)
    for j in (PER_W - 2, PER_W - 1):
        par = j % 4
        p2 = j % 2
        wait_gathers(par)
        wait_write(j - 2, p2)
        compute(par, p2)
        start_write(j, p2)
    wait_write(PER_W - 2, 0)
    wait_write(PER_W - 1, 1)


def kernel(x, embedding_table, possitional_emb):
    xt = x.T.astype(jnp.int32)
    posb = (jnp.broadcast_to(possitional_emb[:, :, None], (L, D, 16))
            .reshape(L * 4, 128))
    xi = _prep(xt)
    out5 = _lookup(xi, embedding_table, posb)
    return out5.transpose(2, 4, 0, 1, 3).reshape(B, L, D)


# speedup vs baseline: 1.2431x; 1.0031x over previous
import functools

import jax
import jax.numpy as jnp
from jax import lax
from jax.experimental import pallas as pl
from jax.experimental.pallas import tpu as pltpu
from jax.experimental.pallas import tpu_sc as plsc

B = 4096
L = 200
D = 32
V = 1000000
NW = 32
CHUNK = 512
NQ = B // CHUNK
ITEMS = L * NQ
PER_W = ITEMS // NW
NG = CHUNK // 128
TBLK = 512
VFULL = V // TBLK
VMAIN = (VFULL // NW) * NW
KMAIN = VMAIN // NW
VTAIL = V - VFULL * TBLK

_mesh = plsc.VectorSubcoreMesh(core_axis_name="c", subcore_axis_name="s")


@functools.partial(
    pl.kernel,
    out_type=jax.ShapeDtypeStruct((L, B // 128, 128), jnp.int32),
    mesh=_mesh,
    scratch_types=[
        pltpu.VMEM((PER_W // 2, 8, 128), jnp.int32),
        pltpu.SemaphoreType.DMA,
        pltpu.SemaphoreType.DMA,
    ],
    compiler_params=pltpu.CompilerParams(use_tc_tiling_on_sc=True,
                                         needs_layout_passes=False),
)
def _prep(xt_hbm, xi_hbm, xb_v, xisem, xosem):
    wid = lax.axis_index("s") * 2 + lax.axis_index("c")

    NB = PER_W // 2
    for i in range(NB):
        m = wid * NB + i
        lt = m // (B // 128)
        c = m % (B // 128)
        pltpu.async_copy(xt_hbm.at[pl.ds(lt * 8, 8), pl.ds(c * 128, 128)],
                         xb_v.at[i], xisem)
    for i in range(NB):
        pltpu.make_async_copy(xt_hbm.at[pl.ds(0, 8), pl.ds(0, 128)],
                              xb_v.at[i], xisem).wait()
    for i in range(NB):
        m = wid * NB + i
        lt = m // (B // 128)
        c = m % (B // 128)
        pltpu.async_copy(xb_v.at[i], xi_hbm.at[pl.ds(lt * 8, 8), c], xosem)
    for i in range(NB):
        m = wid * NB + i
        lt = m // (B // 128)
        c = m % (B // 128)
        pltpu.make_async_copy(xb_v.at[i], xi_hbm.at[pl.ds(lt * 8, 8), c],
                              xosem).wait()


@functools.partial(
    pl.kernel,
    out_type=jax.ShapeDtypeStruct((L, D // 8, B // 128, 8, 128), jnp.float32),
    mesh=_mesh,
    scratch_types=[
        pltpu.VMEM((4, NG, 128), jnp.int32),
        pltpu.VMEM((4, CHUNK, D), jnp.float32),
        pltpu.VMEM((4, 4, 128), jnp.float32),
        pltpu.VMEM((2, D // 8, NG, 8, 128), jnp.float32),
        pltpu.SemaphoreType.DMA,
        pltpu.SemaphoreType.DMA,
        pltpu.SemaphoreType.DMA,
        pltpu.SemaphoreType.DMA,
        pltpu.SemaphoreType.DMA,
        pltpu.SemaphoreType.DMA,
        pltpu.SemaphoreType.DMA,
        pltpu.SemaphoreType.DMA,
        pltpu.SemaphoreType.DMA,
        pltpu.SemaphoreType.DMA,
    ],
    compiler_params=pltpu.CompilerParams(use_tc_tiling_on_sc=False,
                                         needs_layout_passes=False),
)
def _lookup(xi_hbm, trm_hbm, posb_hbm, out_hbm,
            idx_v, rows_v, pos_v, ob_v,
            isem0, isem1, isem2, isem3,
            gsem0, gsem1, gsem2, gsem3, osemA, osemB):
    wid = lax.axis_index("s") * 2 + lax.axis_index("c")
    iota = lax.iota(jnp.int32, 16)
    isem = (isem0, isem1, isem2, isem3)
    gsem = (gsem0, gsem1, gsem2, gsem3)
    osem = (osemA, osemB)

    def lq(j):
        m = wid * PER_W + j
        return m // NQ, m % NQ

    def start_inputs(j, par):
        l, q = lq(j)
        pltpu.async_copy(xi_hbm.at[l, pl.ds(q * NG, NG)], idx_v.at[par],
                         isem[par])
        pltpu.async_copy(posb_hbm.at[pl.ds(l * 4, 4)], pos_v.at[par],
                         isem[par])

    def wait_inputs(par):
        pltpu.make_async_copy(xi_hbm.at[0, pl.ds(0, NG)], idx_v.at[par],
                              isem[par]).wait()
        pltpu.make_async_copy(posb_hbm.at[pl.ds(0, 4)], pos_v.at[par],
                              isem[par]).wait()

    def start_gathers(par):
        for k in range(NG):
            pltpu.async_copy(trm_hbm.at[idx_v.at[par, k]],
                             rows_v.at[par, pl.ds(k * 128, 128)], gsem[par])

    def wait_gathers(par):
        pltpu.make_async_copy(trm_hbm.at[pl.ds(0, CHUNK)], rows_v.at[par],
                              gsem[par]).wait()

    def start_write(j, par):
        l, q = lq(j)
        pltpu.async_copy(ob_v.at[par],
                         out_hbm.at[l, :, pl.ds(q * NG, NG)], osem[par])

    def wait_write(j, par):
        l, q = lq(j)
        pltpu.make_async_copy(ob_v.at[par],
                              out_hbm.at[l, :, pl.ds(q * NG, NG)],
                              osem[par]).wait()

    perm_lo = [(iota - (1 << k)) & 15 for k in range(4)]
    perm_hi = [(iota + (1 << k)) & 15 for k in range(4)]
    masks = [(iota & (1 << k)) == 0 for k in range(4)]
    _dnums = lax.GatherDimensionNumbers(
        offset_dims=(), collapsed_slice_dims=(0,), start_index_map=(0,))

    def _shift(v, perm):
        return lax.gather(v, perm[:, None], _dnums, (1,),
                          mode=lax.GatherScatterMode.PROMISE_IN_BOUNDS)

    def compute(par, p2):
        for dh in range(2):
            def g_body(g, _, dh=dh):
                t0 = g * 16
                cur = [rows_v[par, t0 + i, pl.ds(dh * 16, 16)]
                       for i in range(16)]
                for k in range(4):
                    m = 1 << k
                    nxt = [None] * 16
                    for i in range(16):
                        if i & m == 0:
                            sh = _shift(cur[i + m], perm_lo[k])
                            nxt[i] = jnp.where(masks[k], cur[i], sh)
                        else:
                            sh = _shift(cur[i - m], perm_hi[k])
                            nxt[i] = jnp.where(masks[k], sh, cur[i])
                    cur = nxt
                cp = g // 8
                mm = g % 8
                for j in range(16):
                    d = dh * 16 + j
                    splat = pos_v[par, d // 8, pl.ds((d % 8) * 16, 16)]
                    ob_v[p2, d // 8, cp, d % 8, pl.ds(mm * 16, 16)] = (
                        cur[j] + splat)
                return 0

            lax.fori_loop(0, CHUNK // 16, g_body, 0, unroll=2)

    start_inputs(0, 0)
    start_inputs(1, 1)
    start_inputs(2, 2)
    wait_inputs(0)
    start_gathers(0)
    wait_inputs(1)
    start_gathers(1)

    def quad_body(kk, carry):
        for par in range(4):
            j = kk * 4 + par
            p2 = par % 2
            wait_inputs((par + 2) % 4)
            start_gathers((par + 2) % 4)
            wait_gathers(par)

            @pl.when(j >= 2)
            def _():
                wait_write(j - 2, p2)

            compute(par, p2)
            start_write(j, p2)

            @pl.when(j + 3 < PER_W)
            def _():
                start_inputs(j + 3, (par + 3) % 4)
        return carry

    lax.fori_loop(0, (PER_W - 2) // 4, quad_body, 0)
    for j in (PER_W - 2, PER_W - 1):
        par = j % 4
        p2 = j % 2
        wait_gathers(par)
        wait_write(j - 2, p2)
        compute(par, p2)
        start_write(j, p2)
    wait_write(PER_W - 2, 0)
    wait_write(PER_W - 1, 1)


def kernel(x, embedding_table, possitional_emb):
    xt = x.T.astype(jnp.int32)
    posb = (jnp.broadcast_to(possitional_emb[:, :, None], (L, D, 16))
            .reshape(L * 4, 128))
    xi = _prep(xt)
    out5 = _lookup(xi, embedding_table, posb)
    return out5.transpose(2, 4, 0, 1, 3).reshape(B, L, D)
